# baseline (speedup 1.0000x reference)
import jax
import jax.numpy as jnp
from jax.experimental import pallas as pl

_N = 50000
_D = 100
_LAYERS = 2


def _l2norm(x):
    n = jnp.linalg.norm(x, axis=-1, keepdims=True)
    return x / jnp.maximum(n, 1e-12)


def _combine_body(x0_ref, x1_ref, x2_ref, h1_ref, h2_ref, item_ref, hs_ref):
    def nrm(v):
        n = jnp.sqrt(jnp.sum(v * v, axis=-1, keepdims=True))
        return v / jnp.maximum(n, 1e-12)

    item_ref[...] = (x0_ref[...] + nrm(x1_ref[...]) + nrm(x2_ref[...])) / 3.0
    hs_ref[...] = (nrm(h1_ref[...]) + nrm(h2_ref[...])) / 2.0


def kernel(embedding, edge_index, edge_values, adj, W_item0, W_item1, W_i10, W_i20, channel):
    Ws = [W_item0, W_item1]
    x = embedding
    xs = [x]
    hs_l = []
    for i in range(_LAYERS):
        x = x @ Ws[i].T
        msg = edge_values[:, None] * jnp.take(x, edge_index[1], axis=0)
        x = jax.ops.segment_sum(msg, edge_index[0], num_segments=_N)
        H1 = x @ W_i10.T + x
        H1 = jax.nn.relu(H1)
        H1 = H1 @ W_i20.T
        H1 = jax.nn.softmax(H1, axis=1)
        h = H1.T * adj
        h = h * (1.0 / jnp.sum(h, axis=0))
        h = h @ x
        h = H1 @ h
        x = h + x
        xs.append(x)
        hs_l.append(h)

    BN = 1000
    grid = (_N // BN,)
    spec = pl.BlockSpec((BN, _D), lambda i: (i, 0))
    item, hsum = pl.pallas_call(
        _combine_body,
        grid=grid,
        in_specs=[spec] * 5,
        out_specs=[spec, spec],
        out_shape=[jax.ShapeDtypeStruct((_N, _D), jnp.float32)] * 2,
    )(xs[0], xs[1], xs[2], hs_l[0], hs_l[1])
    return (item, hsum)


# speedup vs baseline: 3.6419x; 3.6419x over previous
import functools

import jax
import jax.numpy as jnp
from jax import lax
from jax.experimental import pallas as pl
from jax.experimental.pallas import tpu as pltpu
from jax.experimental.pallas import tpu_sc as plsc

N = 50000
E = 800000
D = 100
K = 100
DP = 128
NP = 50176
QR = 12544
STRIPE = QR // 16
EPT = E // 16
CH = 512
NFULL = EPT // CH
TAIL = EPT - NFULL * CH
F = 128
CAP = F + 16

BN = 1024
GRID = NP // BN


def _spmm_body(src_hbm, dst_hbm, val_hbm, x_hbm, out_hbm,
               acc, sbuf, dbuf, vbuf, cdst, crow, cval, rowf, rows, ptr_ref):
    core = lax.axis_index("c")
    sub = lax.axis_index("s")
    wid = sub * 2 + core
    iota16 = lax.iota(jnp.int32, 16)
    zero16 = jnp.zeros((16,), jnp.float32)

    e_base = sub * EPT

    def process_chunk():
        @pl.loop(0, F // 16)
        def _cp(k):
            rowf[pl.ds(k * 16, 16)] = crow[pl.ds(k * 16, 16)]
        pltpu.sync_copy(x_hbm.at[cdst.at[pl.ds(0, F)]], rows)
        @pl.loop(0, F // 16)
        def _mul(rg):
            vv = cval[pl.ds(rg * 16, 16)]
            for j in range(16):
                v = jnp.full((16,), vv[j], jnp.float32)
                r = rg * 16 + j
                for u in range(8):
                    rows[r, pl.ds(u * 16, 16)] = rows[r, pl.ds(u * 16, 16)] * v
        pltpu.sync_copy(rows, acc.at[rowf], add=True)

    for q in range(2):
        Q = core * 2 + q
        qbase = Q * QR
        rbase = sub * STRIPE

        @pl.loop(0, F)
        def _zrow(r):
            for u in range(8):
                rows[r, pl.ds(u * 16, 16)] = zero16
        for k in range(STRIPE // F):
            pltpu.sync_copy(rows, acc.at[pl.ds(rbase + k * F, F)])
        rem = STRIPE % F
        if rem:
            pltpu.sync_copy(rows.at[pl.ds(0, rem)],
                            acc.at[pl.ds(rbase + (STRIPE // F) * F, rem)])
        plsc.subcore_barrier()

        ptr_ref[0] = jnp.int32(0)

        def compact_groups(ngroups):
            @pl.loop(0, ngroups)
            def group(g):
                sv = sbuf[pl.ds(g * 16, 16)]
                dv = dbuf[pl.ds(g * 16, 16)]
                vv = vbuf[pl.ds(g * 16, 16)]
                qv = ((sv >= QR).astype(jnp.int32)
                      + (sv >= 2 * QR).astype(jnp.int32)
                      + (sv >= 3 * QR).astype(jnp.int32))
                m = qv == Q
                rv = sv - qbase
                p = ptr_ref[0]
                mi = m.astype(jnp.int32)
                pos = p + plsc.cumsum(mi) - 1
                plsc.store_scatter(cdst, [pos], dv, mask=m)
                plsc.store_scatter(crow, [pos], rv, mask=m)
                plsc.store_scatter(cval, [pos], vv, mask=m)
                p2 = p + jnp.sum(mi)

                @pl.when(p2 >= F)
                def _():
                    process_chunk()
                    for buf in (cdst, crow, cval):
                        t = buf[pl.ds(F, 16)]
                        buf[pl.ds(0, 16)] = t

                ptr_ref[0] = jnp.where(p2 >= F, p2 - F, p2)

        @pl.loop(0, NFULL)
        def _chunk(ci):
            off = e_base + ci * CH
            pltpu.sync_copy(src_hbm.at[pl.ds(off, CH)], sbuf)
            pltpu.sync_copy(dst_hbm.at[pl.ds(off, CH)], dbuf)
            pltpu.sync_copy(val_hbm.at[pl.ds(off, CH)], vbuf)
            compact_groups(CH // 16)

        off = e_base + NFULL * CH
        pltpu.sync_copy(src_hbm.at[pl.ds(off, TAIL)], sbuf.at[pl.ds(0, TAIL)])
        pltpu.sync_copy(dst_hbm.at[pl.ds(off, TAIL)], dbuf.at[pl.ds(0, TAIL)])
        pltpu.sync_copy(val_hbm.at[pl.ds(off, TAIL)], vbuf.at[pl.ds(0, TAIL)])
        compact_groups(TAIL // 16)

        @pl.loop(0, F // 16)
        def _san(g):
            lane = g * 16 + iota16
            m = lane < ptr_ref[0]
            safe = wid * 16 + iota16
            dvv = cdst[pl.ds(g * 16, 16)]
            rvv = crow[pl.ds(g * 16, 16)]
            vvv = cval[pl.ds(g * 16, 16)]
            cdst[pl.ds(g * 16, 16)] = jnp.where(m, dvv, safe)
            crow[pl.ds(g * 16, 16)] = jnp.where(m, rvv, 0)
            cval[pl.ds(g * 16, 16)] = jnp.where(m, vvv, 0.0)
        process_chunk()

        plsc.subcore_barrier()
        pltpu.sync_copy(acc.at[pl.ds(rbase, STRIPE)],
                        out_hbm.at[pl.ds(qbase + rbase, STRIPE)])
        plsc.subcore_barrier()


def _spmm(src, dst, val, xpad):
    mesh = plsc.VectorSubcoreMesh(core_axis_name="c", subcore_axis_name="s")
    f = pl.kernel(
        _spmm_body,
        out_type=jax.ShapeDtypeStruct((NP, DP), jnp.float32),
        mesh=mesh,
        compiler_params=pltpu.CompilerParams(needs_layout_passes=False),
        scratch_types=[
            pltpu.VMEM_SHARED((QR, DP), jnp.float32),
            pltpu.VMEM((CH,), jnp.int32),
            pltpu.VMEM((CH,), jnp.int32),
            pltpu.VMEM((CH,), jnp.float32),
            pltpu.VMEM((CAP,), jnp.int32),
            pltpu.VMEM((CAP,), jnp.int32),
            pltpu.VMEM((CAP,), jnp.float32),
            pltpu.VMEM((F,), jnp.int32),
            pltpu.VMEM((F, DP), jnp.float32),
            pltpu.SMEM((1,), jnp.int32),
        ],
    )
    return f(src, dst, val, xpad)


def _l2n(x):
    n = jnp.sqrt(jnp.sum(x * x, axis=1, keepdims=True))
    return x / jnp.maximum(n, 1e-12)


def _linear_body(x_ref, w_ref, o_ref):
    o_ref[...] = jnp.dot(x_ref[...], w_ref[...],
                         preferred_element_type=jnp.float32)


def _linear(x, wt):
    return pl.pallas_call(
        _linear_body,
        grid=(GRID,),
        in_specs=[pl.BlockSpec((BN, DP), lambda i: (i, 0)),
                  pl.BlockSpec((DP, DP), lambda i: (0, 0))],
        out_specs=pl.BlockSpec((BN, DP), lambda i: (i, 0)),
        out_shape=jax.ShapeDtypeStruct((NP, DP), jnp.float32),
    )(x, wt)


def _phase_a_body(xr_ref, w10t_ref, w20t_ref, adjt_ref, h1_ref, m_ref, macc_ref):
    i = pl.program_id(0)
    x = xr_ref[...]
    t = jnp.maximum(jnp.dot(x, w10t_ref[...],
                            preferred_element_type=jnp.float32), 0.0)
    logits = jnp.dot(t, w20t_ref[...], preferred_element_type=jnp.float32)
    lanes = lax.broadcasted_iota(jnp.int32, (1, DP), 1)
    logits = jnp.where(lanes < K, logits, -1e30)
    mx = jnp.max(logits, axis=1, keepdims=True)
    e = jnp.exp(logits - mx)
    h1 = e / jnp.sum(e, axis=1, keepdims=True)
    h1_ref[...] = h1
    a = h1 * adjt_ref[...]
    s = jnp.sum(a, axis=1, keepdims=True)
    g = a / jnp.maximum(s, 1e-30)
    mblk = lax.dot_general(g, x, (((0,), (0,)), ((), ())),
                           preferred_element_type=jnp.float32)

    @pl.when(i == 0)
    def _():
        macc_ref[...] = mblk

    @pl.when(i > 0)
    def _():
        macc_ref[...] += mblk

    @pl.when(i == GRID - 1)
    def _():
        m_ref[...] = macc_ref[...]


def _phase_a(xr, w10t, w20t, adjt):
    return pl.pallas_call(
        _phase_a_body,
        grid=(GRID,),
        in_specs=[pl.BlockSpec((BN, DP), lambda i: (i, 0)),
                  pl.BlockSpec((DP, DP), lambda i: (0, 0)),
                  pl.BlockSpec((DP, DP), lambda i: (0, 0)),
                  pl.BlockSpec((BN, DP), lambda i: (i, 0))],
        out_specs=[pl.BlockSpec((BN, DP), lambda i: (i, 0)),
                   pl.BlockSpec((DP, DP), lambda i: (0, 0))],
        out_shape=[jax.ShapeDtypeStruct((NP, DP), jnp.float32),
                   jax.ShapeDtypeStruct((DP, DP), jnp.float32)],
        scratch_shapes=[pltpu.VMEM((DP, DP), jnp.float32)],
    )(xr, w10t, w20t, adjt)


def _phase_b1_body(h1_ref, m_ref, xr_ref, emb_ref, w1t_ref,
                   item1_ref, hn1_ref, xw2_ref):
    h = jnp.dot(h1_ref[...], m_ref[...], preferred_element_type=jnp.float32)
    x1 = h + xr_ref[...]
    item1_ref[...] = emb_ref[...] + _l2n(x1)
    hn1_ref[...] = _l2n(h)
    xw2_ref[...] = jnp.dot(x1, w1t_ref[...], preferred_element_type=jnp.float32)


def _phase_b1(h1, m, xr, emb, w1t):
    blk = pl.BlockSpec((BN, DP), lambda i: (i, 0))
    return pl.pallas_call(
        _phase_b1_body,
        grid=(GRID,),
        in_specs=[blk, pl.BlockSpec((DP, DP), lambda i: (0, 0)), blk, blk,
                  pl.BlockSpec((DP, DP), lambda i: (0, 0))],
        out_specs=[blk, blk, blk],
        out_shape=[jax.ShapeDtypeStruct((NP, DP), jnp.float32)] * 3,
    )(h1, m, xr, emb, w1t)


def _phase_b2_body(h1_ref, m_ref, xr_ref, item1_ref, hn1_ref,
                   item_ref, hs_ref):
    h = jnp.dot(h1_ref[...], m_ref[...], preferred_element_type=jnp.float32)
    x2 = h + xr_ref[...]
    item_ref[...] = (item1_ref[...] + _l2n(x2)) * (1.0 / 3.0)
    hs_ref[...] = (hn1_ref[...] + _l2n(h)) * 0.5


def _phase_b2(h1, m, xr, item1, hn1):
    blk = pl.BlockSpec((BN, DP), lambda i: (i, 0))
    return pl.pallas_call(
        _phase_b2_body,
        grid=(GRID,),
        in_specs=[blk, pl.BlockSpec((DP, DP), lambda i: (0, 0)), blk, blk, blk],
        out_specs=[blk, blk],
        out_shape=[jax.ShapeDtypeStruct((NP, DP), jnp.float32)] * 2,
    )(h1, m, xr, item1, hn1)


def _pad2(a, rows, cols):
    return jnp.zeros((rows, cols), a.dtype).at[:a.shape[0], :a.shape[1]].set(a)


def kernel(embedding, edge_index, edge_values, adj, W_item0, W_item1,
           W_i10, W_i20, channel):
    src = edge_index[0]
    dst = edge_index[1]
    emb = _pad2(embedding, NP, DP)
    adjt = _pad2(adj.T, NP, DP)
    w0t = _pad2(W_item0.T, DP, DP)
    w1t = _pad2(W_item1.T, DP, DP)
    w10t = _pad2(W_i10.T + jnp.eye(D, dtype=jnp.float32), DP, DP)
    w20t = _pad2(W_i20.T, DP, DP)

    xw1 = _linear(emb, w0t)
    xr1 = _spmm(src, dst, edge_values, xw1)
    h1_1, m1 = _phase_a(xr1, w10t, w20t, adjt)
    item1, hn1, xw2 = _phase_b1(h1_1, m1, xr1, emb, w1t)
    xr2 = _spmm(src, dst, edge_values, xw2)
    h1_2, m2 = _phase_a(xr2, w10t, w20t, adjt)
    item, hs = _phase_b2(h1_2, m2, xr2, item1, hn1)
    return (item[:N, :D], hs[:N, :D])


# speedup vs baseline: 4.7147x; 1.2946x over previous
import functools

import jax
import jax.numpy as jnp
from jax import lax
from jax.experimental import pallas as pl
from jax.experimental.pallas import tpu as pltpu
from jax.experimental.pallas import tpu_sc as plsc

N = 50000
E = 800000
D = 100
K = 100
DP = 128
NP = 50176
QR = 12544
STRIPE = QR // 16
EPT = E // 16
CH = 1024
NFULL = EPT // CH
NPAIR = NFULL // 2
TAIL = EPT - NFULL * CH
F = 128
CAP = F + 16

BN = 1024
GRID = NP // BN


def _spmm_body(src_hbm, dst_hbm, val_hbm, x_hbm, out_hbm,
               acc, sbufa, dbufa, vbufa, sbufb, dbufb, vbufb,
               cdst, crow, cval, rowf, rows, ptr_ref, sema, semb):
    core = lax.axis_index("c")
    sub = lax.axis_index("s")
    wid = sub * 2 + core
    iota16 = lax.iota(jnp.int32, 16)
    zero16 = jnp.zeros((16,), jnp.float32)

    e_base = sub * EPT

    def process_chunk():
        @pl.loop(0, F // 16)
        def _cp(k):
            rowf[pl.ds(k * 16, 16)] = crow[pl.ds(k * 16, 16)]
        pltpu.sync_copy(x_hbm.at[cdst.at[pl.ds(0, F)]], rows)
        @pl.loop(0, F // 16)
        def _mul(rg):
            vv = cval[pl.ds(rg * 16, 16)]
            for j in range(16):
                v = jnp.full((16,), vv[j], jnp.float32)
                r = rg * 16 + j
                for u in range(8):
                    rows[r, pl.ds(u * 16, 16)] = rows[r, pl.ds(u * 16, 16)] * v
        pltpu.sync_copy(rows, acc.at[rowf], add=True)

    for q in range(2):
        Q = core * 2 + q
        qbase = Q * QR
        rbase = sub * STRIPE

        @pl.loop(0, F)
        def _zrow(r):
            for u in range(8):
                rows[r, pl.ds(u * 16, 16)] = zero16
        for k in range(STRIPE // F):
            pltpu.sync_copy(rows, acc.at[pl.ds(rbase + k * F, F)])
        rem = STRIPE % F
        if rem:
            pltpu.sync_copy(rows.at[pl.ds(0, rem)],
                            acc.at[pl.ds(rbase + (STRIPE // F) * F, rem)])
        plsc.subcore_barrier()

        ptr_ref[0] = jnp.int32(0)

        def compact_groups(ngroups, sbuf, dbuf, vbuf):
            @pl.loop(0, ngroups)
            def group(g):
                sv = sbuf[pl.ds(g * 16, 16)]
                dv = dbuf[pl.ds(g * 16, 16)]
                vv = vbuf[pl.ds(g * 16, 16)]
                qv = ((sv >= QR).astype(jnp.int32)
                      + (sv >= 2 * QR).astype(jnp.int32)
                      + (sv >= 3 * QR).astype(jnp.int32))
                m = qv == Q
                rv = sv - qbase
                p = ptr_ref[0]
                mi = m.astype(jnp.int32)
                pos = p + plsc.cumsum(mi) - 1
                plsc.store_scatter(cdst, [pos], dv, mask=m)
                plsc.store_scatter(crow, [pos], rv, mask=m)
                plsc.store_scatter(cval, [pos], vv, mask=m)
                p2 = p + jnp.sum(mi)

                @pl.when(p2 >= F)
                def _():
                    process_chunk()
                    for buf in (cdst, crow, cval):
                        t = buf[pl.ds(F, 16)]
                        buf[pl.ds(0, 16)] = t

                ptr_ref[0] = jnp.where(p2 >= F, p2 - F, p2)

        def issue(off, sb, db, vb, sem):
            pltpu.async_copy(src_hbm.at[pl.ds(off, CH)], sb, sem)
            pltpu.async_copy(dst_hbm.at[pl.ds(off, CH)], db, sem)
            pltpu.async_copy(val_hbm.at[pl.ds(off, CH)], vb, sem)

        def drain(sb, db, vb, sem):
            pltpu.make_async_copy(src_hbm.at[pl.ds(0, CH)], sb, sem).wait()
            pltpu.make_async_copy(dst_hbm.at[pl.ds(0, CH)], db, sem).wait()
            pltpu.make_async_copy(val_hbm.at[pl.ds(0, CH)], vb, sem).wait()

        issue(e_base, sbufa, dbufa, vbufa, sema)

        @pl.loop(0, NPAIR)
        def _pair(k):
            ci = k * 2
            issue(e_base + (ci + 1) * CH, sbufb, dbufb, vbufb, semb)
            drain(sbufa, dbufa, vbufa, sema)
            compact_groups(CH // 16, sbufa, dbufa, vbufa)

            @pl.when(k < NPAIR - 1)
            def _():
                issue(e_base + (ci + 2) * CH, sbufa, dbufa, vbufa, sema)

            drain(sbufb, dbufb, vbufb, semb)
            compact_groups(CH // 16, sbufb, dbufb, vbufb)

        off = e_base + NFULL * CH
        pltpu.sync_copy(src_hbm.at[pl.ds(off, TAIL)], sbufa.at[pl.ds(0, TAIL)])
        pltpu.sync_copy(dst_hbm.at[pl.ds(off, TAIL)], dbufa.at[pl.ds(0, TAIL)])
        pltpu.sync_copy(val_hbm.at[pl.ds(off, TAIL)], vbufa.at[pl.ds(0, TAIL)])
        compact_groups(TAIL // 16, sbufa, dbufa, vbufa)

        @pl.loop(0, F // 16)
        def _san(g):
            lane = g * 16 + iota16
            m = lane < ptr_ref[0]
            safe = wid * 16 + iota16
            dvv = cdst[pl.ds(g * 16, 16)]
            rvv = crow[pl.ds(g * 16, 16)]
            vvv = cval[pl.ds(g * 16, 16)]
            cdst[pl.ds(g * 16, 16)] = jnp.where(m, dvv, safe)
            crow[pl.ds(g * 16, 16)] = jnp.where(m, rvv, 0)
            cval[pl.ds(g * 16, 16)] = jnp.where(m, vvv, 0.0)
        process_chunk()

        plsc.subcore_barrier()
        pltpu.sync_copy(acc.at[pl.ds(rbase, STRIPE)],
                        out_hbm.at[pl.ds(qbase + rbase, STRIPE)])
        plsc.subcore_barrier()


def _spmm(src, dst, val, xpad):
    mesh = plsc.VectorSubcoreMesh(core_axis_name="c", subcore_axis_name="s")
    f = pl.kernel(
        _spmm_body,
        out_type=jax.ShapeDtypeStruct((NP, DP), jnp.float32),
        mesh=mesh,
        compiler_params=pltpu.CompilerParams(needs_layout_passes=False),
        scratch_types=[
            pltpu.VMEM_SHARED((QR, DP), jnp.float32),
            pltpu.VMEM((CH,), jnp.int32),
            pltpu.VMEM((CH,), jnp.int32),
            pltpu.VMEM((CH,), jnp.float32),
            pltpu.VMEM((CH,), jnp.int32),
            pltpu.VMEM((CH,), jnp.int32),
            pltpu.VMEM((CH,), jnp.float32),
            pltpu.VMEM((CAP,), jnp.int32),
            pltpu.VMEM((CAP,), jnp.int32),
            pltpu.VMEM((CAP,), jnp.float32),
            pltpu.VMEM((F,), jnp.int32),
            pltpu.VMEM((F, DP), jnp.float32),
            pltpu.SMEM((1,), jnp.int32),
            pltpu.SemaphoreType.DMA,
            pltpu.SemaphoreType.DMA,
        ],
    )
    return f(src, dst, val, xpad)


def _l2n(x):
    n = jnp.sqrt(jnp.sum(x * x, axis=1, keepdims=True))
    return x / jnp.maximum(n, 1e-12)


def _linear_body(x_ref, w_ref, o_ref):
    o_ref[...] = jnp.dot(x_ref[...], w_ref[...],
                         preferred_element_type=jnp.float32)


def _linear(x, wt):
    return pl.pallas_call(
        _linear_body,
        grid=(GRID,),
        in_specs=[pl.BlockSpec((BN, DP), lambda i: (i, 0)),
                  pl.BlockSpec((DP, DP), lambda i: (0, 0))],
        out_specs=pl.BlockSpec((BN, DP), lambda i: (i, 0)),
        out_shape=jax.ShapeDtypeStruct((NP, DP), jnp.float32),
    )(x, wt)


def _phase_a_body(xr_ref, w10t_ref, w20t_ref, adjt_ref, h1_ref, m_ref, macc_ref):
    i = pl.program_id(0)
    x = xr_ref[...]
    t = jnp.maximum(jnp.dot(x, w10t_ref[...],
                            preferred_element_type=jnp.float32), 0.0)
    logits = jnp.dot(t, w20t_ref[...], preferred_element_type=jnp.float32)
    lanes = lax.broadcasted_iota(jnp.int32, (1, DP), 1)
    logits = jnp.where(lanes < K, logits, -1e30)
    mx = jnp.max(logits, axis=1, keepdims=True)
    e = jnp.exp(logits - mx)
    h1 = e / jnp.sum(e, axis=1, keepdims=True)
    h1_ref[...] = h1
    a = h1 * adjt_ref[...]
    s = jnp.sum(a, axis=1, keepdims=True)
    g = a / jnp.maximum(s, 1e-30)
    mblk = lax.dot_general(g, x, (((0,), (0,)), ((), ())),
                           preferred_element_type=jnp.float32)

    @pl.when(i == 0)
    def _():
        macc_ref[...] = mblk

    @pl.when(i > 0)
    def _():
        macc_ref[...] += mblk

    @pl.when(i == GRID - 1)
    def _():
        m_ref[...] = macc_ref[...]


def _phase_a(xr, w10t, w20t, adjt):
    return pl.pallas_call(
        _phase_a_body,
        grid=(GRID,),
        in_specs=[pl.BlockSpec((BN, DP), lambda i: (i, 0)),
                  pl.BlockSpec((DP, DP), lambda i: (0, 0)),
                  pl.BlockSpec((DP, DP), lambda i: (0, 0)),
                  pl.BlockSpec((BN, DP), lambda i: (i, 0))],
        out_specs=[pl.BlockSpec((BN, DP), lambda i: (i, 0)),
                   pl.BlockSpec((DP, DP), lambda i: (0, 0))],
        out_shape=[jax.ShapeDtypeStruct((NP, DP), jnp.float32),
                   jax.ShapeDtypeStruct((DP, DP), jnp.float32)],
        scratch_shapes=[pltpu.VMEM((DP, DP), jnp.float32)],
    )(xr, w10t, w20t, adjt)


def _phase_b1_body(h1_ref, m_ref, xr_ref, emb_ref, w1t_ref,
                   item1_ref, hn1_ref, xw2_ref):
    h = jnp.dot(h1_ref[...], m_ref[...], preferred_element_type=jnp.float32)
    x1 = h + xr_ref[...]
    item1_ref[...] = emb_ref[...] + _l2n(x1)
    hn1_ref[...] = _l2n(h)
    xw2_ref[...] = jnp.dot(x1, w1t_ref[...], preferred_element_type=jnp.float32)


def _phase_b1(h1, m, xr, emb, w1t):
    blk = pl.BlockSpec((BN, DP), lambda i: (i, 0))
    return pl.pallas_call(
        _phase_b1_body,
        grid=(GRID,),
        in_specs=[blk, pl.BlockSpec((DP, DP), lambda i: (0, 0)), blk, blk,
                  pl.BlockSpec((DP, DP), lambda i: (0, 0))],
        out_specs=[blk, blk, blk],
        out_shape=[jax.ShapeDtypeStruct((NP, DP), jnp.float32)] * 3,
    )(h1, m, xr, emb, w1t)


def _phase_b2_body(h1_ref, m_ref, xr_ref, item1_ref, hn1_ref,
                   item_ref, hs_ref):
    h = jnp.dot(h1_ref[...], m_ref[...], preferred_element_type=jnp.float32)
    x2 = h + xr_ref[...]
    item_ref[...] = (item1_ref[...] + _l2n(x2)) * (1.0 / 3.0)
    hs_ref[...] = (hn1_ref[...] + _l2n(h)) * 0.5


def _phase_b2(h1, m, xr, item1, hn1):
    blk = pl.BlockSpec((BN, DP), lambda i: (i, 0))
    return pl.pallas_call(
        _phase_b2_body,
        grid=(GRID,),
        in_specs=[blk, pl.BlockSpec((DP, DP), lambda i: (0, 0)), blk, blk, blk],
        out_specs=[blk, blk],
        out_shape=[jax.ShapeDtypeStruct((NP, DP), jnp.float32)] * 2,
    )(h1, m, xr, item1, hn1)


def _pad2(a, rows, cols):
    return jnp.zeros((rows, cols), a.dtype).at[:a.shape[0], :a.shape[1]].set(a)


def kernel(embedding, edge_index, edge_values, adj, W_item0, W_item1,
           W_i10, W_i20, channel):
    src = edge_index[0]
    dst = edge_index[1]
    emb = _pad2(embedding, NP, DP)
    adjt = _pad2(adj.T, NP, DP)
    w0t = _pad2(W_item0.T, DP, DP)
    w1t = _pad2(W_item1.T, DP, DP)
    w10t = _pad2(W_i10.T + jnp.eye(D, dtype=jnp.float32), DP, DP)
    w20t = _pad2(W_i20.T, DP, DP)

    xw1 = _linear(emb, w0t)
    xr1 = _spmm(src, dst, edge_values, xw1)
    h1_1, m1 = _phase_a(xr1, w10t, w20t, adjt)
    item1, hn1, xw2 = _phase_b1(h1_1, m1, xr1, emb, w1t)
    xr2 = _spmm(src, dst, edge_values, xw2)
    h1_2, m2 = _phase_a(xr2, w10t, w20t, adjt)
    item, hs = _phase_b2(h1_2, m2, xr2, item1, hn1)
    return (item[:N, :D], hs[:N, :D])


# speedup vs baseline: 5.9515x; 1.2623x over previous
import functools

import jax
import jax.numpy as jnp
from jax import lax
from jax.experimental import pallas as pl
from jax.experimental.pallas import tpu as pltpu
from jax.experimental.pallas import tpu_sc as plsc

N = 50000
E = 800000
D = 100
K = 100
DP = 128
NP = 50176
QR = 12544
STRIPE = QR // 16
EPT = E // 16
CH = 1024
NFULL = EPT // CH
NPAIR = NFULL // 2
TAIL = EPT - NFULL * CH
F = 80
CAP = F + 16

BN = 1024
GRID = NP // BN


def _spmm_body(src_hbm, dst_hbm, val_hbm, x_hbm, out_hbm,
               acc, sbufa, dbufa, vbufa, sbufb, dbufb, vbufb,
               cdst, crow, cval,
               dstf0, valf0, rowf0, rows0,
               dstf1, valf1, rowf1, rows1,
               ptr_ref, flg, sema, semb, gs0, gs1, ss0, ss1):
    core = lax.axis_index("c")
    sub = lax.axis_index("s")
    wid = sub * 2 + core
    iota16 = lax.iota(jnp.int32, 16)
    zero16 = jnp.zeros((16,), jnp.float32)

    e_base = sub * EPT
    sets = ((dstf0, valf0, rowf0, rows0, gs0, ss0),
            (dstf1, valf1, rowf1, rows1, gs1, ss1))

    def mul_rows(valf, rows):
        @pl.loop(0, F // 16)
        def _mul(rg):
            vv = valf[pl.ds(rg * 16, 16)]
            for j in range(16):
                v = jnp.full((16,), vv[j], jnp.float32)
                r = rg * 16 + j
                for u in range(8):
                    rows[r, pl.ds(u * 16, 16)] = rows[r, pl.ds(u * 16, 16)] * v

    def finish(t):
        dstf, valf, rowf, rows, gsem, ssem = sets[t]

        @pl.when(flg[1 + t] == 1)
        def _():
            pltpu.make_async_copy(x_hbm.at[dstf], rows, gsem).wait()
            mul_rows(valf, rows)
            pltpu.async_copy(rows, acc.at[rowf], ssem, add=True)
            flg[1 + t] = jnp.int32(0)
            flg[3 + t] = jnp.int32(1)

    def wait_scatter(t):
        dstf, valf, rowf, rows, gsem, ssem = sets[t]

        @pl.when(flg[3 + t] == 1)
        def _():
            pltpu.make_async_copy(rows, acc.at[rowf], ssem).wait()
            flg[3 + t] = jnp.int32(0)

    def start(t):
        dstf, valf, rowf, rows, gsem, ssem = sets[t]

        @pl.loop(0, F // 16)
        def _cp(k):
            dstf[pl.ds(k * 16, 16)] = cdst[pl.ds(k * 16, 16)]
            valf[pl.ds(k * 16, 16)] = cval[pl.ds(k * 16, 16)]
            rowf[pl.ds(k * 16, 16)] = crow[pl.ds(k * 16, 16)]
        pltpu.async_copy(x_hbm.at[dstf], rows, gsem)
        flg[1 + t] = jnp.int32(1)

    for q in range(2):
        Q = core * 2 + q
        qbase = Q * QR
        rbase = sub * STRIPE

        @pl.loop(0, F)
        def _zrow(r):
            for u in range(8):
                rows0[r, pl.ds(u * 16, 16)] = zero16
        for k in range(STRIPE // F):
            pltpu.sync_copy(rows0, acc.at[pl.ds(rbase + k * F, F)])
        rem = STRIPE % F
        if rem:
            pltpu.sync_copy(rows0.at[pl.ds(0, rem)],
                            acc.at[pl.ds(rbase + (STRIPE // F) * F, rem)])
        plsc.subcore_barrier()

        ptr_ref[0] = jnp.int32(0)
        for i in range(5):
            flg[i] = jnp.int32(0)

        def fire():
            a = flg[0]

            @pl.when(a == 0)
            def _():
                finish(1)
                wait_scatter(0)
                start(0)

            @pl.when(a == 1)
            def _():
                finish(0)
                wait_scatter(1)
                start(1)

            flg[0] = 1 - a
            for buf in (cdst, crow, cval):
                t_ = buf[pl.ds(F, 16)]
                buf[pl.ds(0, 16)] = t_

        def compact_groups(ngroups, sbuf, dbuf, vbuf):
            @pl.loop(0, ngroups)
            def group(g):
                sv = sbuf[pl.ds(g * 16, 16)]
                dv = dbuf[pl.ds(g * 16, 16)]
                vv = vbuf[pl.ds(g * 16, 16)]
                qv = ((sv >= QR).astype(jnp.int32)
                      + (sv >= 2 * QR).astype(jnp.int32)
                      + (sv >= 3 * QR).astype(jnp.int32))
                m = qv == Q
                rv = sv - qbase
                p = ptr_ref[0]
                mi = m.astype(jnp.int32)
                pos = p + plsc.cumsum(mi) - 1
                plsc.store_scatter(cdst, [pos], dv, mask=m)
                plsc.store_scatter(crow, [pos], rv, mask=m)
                plsc.store_scatter(cval, [pos], vv, mask=m)
                p2 = p + jnp.sum(mi)

                @pl.when(p2 >= F)
                def _():
                    fire()

                ptr_ref[0] = jnp.where(p2 >= F, p2 - F, p2)

        def issue(off, sb, db, vb, sem):
            pltpu.async_copy(src_hbm.at[pl.ds(off, CH)], sb, sem)
            pltpu.async_copy(dst_hbm.at[pl.ds(off, CH)], db, sem)
            pltpu.async_copy(val_hbm.at[pl.ds(off, CH)], vb, sem)

        def drain(sb, db, vb, sem):
            pltpu.make_async_copy(src_hbm.at[pl.ds(0, CH)], sb, sem).wait()
            pltpu.make_async_copy(dst_hbm.at[pl.ds(0, CH)], db, sem).wait()
            pltpu.make_async_copy(val_hbm.at[pl.ds(0, CH)], vb, sem).wait()

        issue(e_base, sbufa, dbufa, vbufa, sema)

        @pl.loop(0, NPAIR)
        def _pair(k):
            ci = k * 2
            issue(e_base + (ci + 1) * CH, sbufb, dbufb, vbufb, semb)
            drain(sbufa, dbufa, vbufa, sema)
            compact_groups(CH // 16, sbufa, dbufa, vbufa)

            @pl.when(k < NPAIR - 1)
            def _():
                issue(e_base + (ci + 2) * CH, sbufa, dbufa, vbufa, sema)

            drain(sbufb, dbufb, vbufb, semb)
            compact_groups(CH // 16, sbufb, dbufb, vbufb)

        off = e_base + NFULL * CH
        pltpu.sync_copy(src_hbm.at[pl.ds(off, TAIL)], sbufa.at[pl.ds(0, TAIL)])
        pltpu.sync_copy(dst_hbm.at[pl.ds(off, TAIL)], dbufa.at[pl.ds(0, TAIL)])
        pltpu.sync_copy(val_hbm.at[pl.ds(off, TAIL)], vbufa.at[pl.ds(0, TAIL)])
        compact_groups(TAIL // 16, sbufa, dbufa, vbufa)

        finish(0)
        finish(1)
        wait_scatter(0)
        wait_scatter(1)

        @pl.loop(0, F // 16)
        def _san(g):
            lane = g * 16 + iota16
            m = lane < ptr_ref[0]
            safe = wid * 16 + iota16
            dvv = cdst[pl.ds(g * 16, 16)]
            rvv = crow[pl.ds(g * 16, 16)]
            vvv = cval[pl.ds(g * 16, 16)]
            cdst[pl.ds(g * 16, 16)] = jnp.where(m, dvv, safe)
            crow[pl.ds(g * 16, 16)] = jnp.where(m, rvv, 0)
            cval[pl.ds(g * 16, 16)] = jnp.where(m, vvv, 0.0)
        start(0)
        finish(0)
        wait_scatter(0)

        plsc.subcore_barrier()
        pltpu.sync_copy(acc.at[pl.ds(rbase, STRIPE)],
                        out_hbm.at[pl.ds(qbase + rbase, STRIPE)])
        plsc.subcore_barrier()


def _spmm(src, dst, val, xpad):
    mesh = plsc.VectorSubcoreMesh(core_axis_name="c", subcore_axis_name="s")
    f = pl.kernel(
        _spmm_body,
        out_type=jax.ShapeDtypeStruct((NP, DP), jnp.float32),
        mesh=mesh,
        compiler_params=pltpu.CompilerParams(needs_layout_passes=False),
        scratch_types=[
            pltpu.VMEM_SHARED((QR, DP), jnp.float32),
            pltpu.VMEM((CH,), jnp.int32),
            pltpu.VMEM((CH,), jnp.int32),
            pltpu.VMEM((CH,), jnp.float32),
            pltpu.VMEM((CH,), jnp.int32),
            pltpu.VMEM((CH,), jnp.int32),
            pltpu.VMEM((CH,), jnp.float32),
            pltpu.VMEM((CAP,), jnp.int32),
            pltpu.VMEM((CAP,), jnp.int32),
            pltpu.VMEM((CAP,), jnp.float32),
            pltpu.VMEM((F,), jnp.int32),
            pltpu.VMEM((F,), jnp.float32),
            pltpu.VMEM((F,), jnp.int32),
            pltpu.VMEM((F, DP), jnp.float32),
            pltpu.VMEM((F,), jnp.int32),
            pltpu.VMEM((F,), jnp.float32),
            pltpu.VMEM((F,), jnp.int32),
            pltpu.VMEM((F, DP), jnp.float32),
            pltpu.SMEM((1,), jnp.int32),
            pltpu.SMEM((5,), jnp.int32),
            pltpu.SemaphoreType.DMA,
            pltpu.SemaphoreType.DMA,
            pltpu.SemaphoreType.DMA,
            pltpu.SemaphoreType.DMA,
            pltpu.SemaphoreType.DMA,
            pltpu.SemaphoreType.DMA,
        ],
    )
    return f(src, dst, val, xpad)


def _l2n(x):
    n = jnp.sqrt(jnp.sum(x * x, axis=1, keepdims=True))
    return x / jnp.maximum(n, 1e-12)


def _linear_body(x_ref, w_ref, o_ref):
    o_ref[...] = jnp.dot(x_ref[...], w_ref[...],
                         preferred_element_type=jnp.float32)


def _linear(x, wt):
    return pl.pallas_call(
        _linear_body,
        grid=(GRID,),
        in_specs=[pl.BlockSpec((BN, DP), lambda i: (i, 0)),
                  pl.BlockSpec((DP, DP), lambda i: (0, 0))],
        out_specs=pl.BlockSpec((BN, DP), lambda i: (i, 0)),
        out_shape=jax.ShapeDtypeStruct((NP, DP), jnp.float32),
    )(x, wt)


def _phase_a_body(xr_ref, w10t_ref, w20t_ref, adjt_ref, h1_ref, m_ref, macc_ref):
    i = pl.program_id(0)
    x = xr_ref[...]
    t = jnp.maximum(jnp.dot(x, w10t_ref[...],
                            preferred_element_type=jnp.float32), 0.0)
    logits = jnp.dot(t, w20t_ref[...], preferred_element_type=jnp.float32)
    lanes = lax.broadcasted_iota(jnp.int32, (1, DP), 1)
    logits = jnp.where(lanes < K, logits, -1e30)
    mx = jnp.max(logits, axis=1, keepdims=True)
    e = jnp.exp(logits - mx)
    h1 = e / jnp.sum(e, axis=1, keepdims=True)
    h1_ref[...] = h1
    a = h1 * adjt_ref[...]
    s = jnp.sum(a, axis=1, keepdims=True)
    g = a / jnp.maximum(s, 1e-30)
    mblk = lax.dot_general(g, x, (((0,), (0,)), ((), ())),
                           preferred_element_type=jnp.float32)

    @pl.when(i == 0)
    def _():
        macc_ref[...] = mblk

    @pl.when(i > 0)
    def _():
        macc_ref[...] += mblk

    @pl.when(i == GRID - 1)
    def _():
        m_ref[...] = macc_ref[...]


def _phase_a(xr, w10t, w20t, adjt):
    return pl.pallas_call(
        _phase_a_body,
        grid=(GRID,),
        in_specs=[pl.BlockSpec((BN, DP), lambda i: (i, 0)),
                  pl.BlockSpec((DP, DP), lambda i: (0, 0)),
                  pl.BlockSpec((DP, DP), lambda i: (0, 0)),
                  pl.BlockSpec((BN, DP), lambda i: (i, 0))],
        out_specs=[pl.BlockSpec((BN, DP), lambda i: (i, 0)),
                   pl.BlockSpec((DP, DP), lambda i: (0, 0))],
        out_shape=[jax.ShapeDtypeStruct((NP, DP), jnp.float32),
                   jax.ShapeDtypeStruct((DP, DP), jnp.float32)],
        scratch_shapes=[pltpu.VMEM((DP, DP), jnp.float32)],
    )(xr, w10t, w20t, adjt)


def _phase_b1_body(h1_ref, m_ref, xr_ref, emb_ref, w1t_ref,
                   item1_ref, hn1_ref, xw2_ref):
    h = jnp.dot(h1_ref[...], m_ref[...], preferred_element_type=jnp.float32)
    x1 = h + xr_ref[...]
    item1_ref[...] = emb_ref[...] + _l2n(x1)
    hn1_ref[...] = _l2n(h)
    xw2_ref[...] = jnp.dot(x1, w1t_ref[...], preferred_element_type=jnp.float32)


def _phase_b1(h1, m, xr, emb, w1t):
    blk = pl.BlockSpec((BN, DP), lambda i: (i, 0))
    return pl.pallas_call(
        _phase_b1_body,
        grid=(GRID,),
        in_specs=[blk, pl.BlockSpec((DP, DP), lambda i: (0, 0)), blk, blk,
                  pl.BlockSpec((DP, DP), lambda i: (0, 0))],
        out_specs=[blk, blk, blk],
        out_shape=[jax.ShapeDtypeStruct((NP, DP), jnp.float32)] * 3,
    )(h1, m, xr, emb, w1t)


def _phase_b2_body(h1_ref, m_ref, xr_ref, item1_ref, hn1_ref,
                   item_ref, hs_ref):
    h = jnp.dot(h1_ref[...], m_ref[...], preferred_element_type=jnp.float32)
    x2 = h + xr_ref[...]
    item_ref[...] = (item1_ref[...] + _l2n(x2)) * (1.0 / 3.0)
    hs_ref[...] = (hn1_ref[...] + _l2n(h)) * 0.5


def _phase_b2(h1, m, xr, item1, hn1):
    blk = pl.BlockSpec((BN, DP), lambda i: (i, 0))
    return pl.pallas_call(
        _phase_b2_body,
        grid=(GRID,),
        in_specs=[blk, pl.BlockSpec((DP, DP), lambda i: (0, 0)), blk, blk, blk],
        out_specs=[blk, blk],
        out_shape=[jax.ShapeDtypeStruct((NP, DP), jnp.float32)] * 2,
    )(h1, m, xr, item1, hn1)


def _pad2(a, rows, cols):
    return jnp.zeros((rows, cols), a.dtype).at[:a.shape[0], :a.shape[1]].set(a)


def kernel(embedding, edge_index, edge_values, adj, W_item0, W_item1,
           W_i10, W_i20, channel):
    src = edge_index[0]
    dst = edge_index[1]
    emb = _pad2(embedding, NP, DP)
    adjt = _pad2(adj.T, NP, DP)
    w0t = _pad2(W_item0.T, DP, DP)
    w1t = _pad2(W_item1.T, DP, DP)
    w10t = _pad2(W_i10.T + jnp.eye(D, dtype=jnp.float32), DP, DP)
    w20t = _pad2(W_i20.T, DP, DP)

    xw1 = _linear(emb, w0t)
    xr1 = _spmm(src, dst, edge_values, xw1)
    h1_1, m1 = _phase_a(xr1, w10t, w20t, adjt)
    item1, hn1, xw2 = _phase_b1(h1_1, m1, xr1, emb, w1t)
    xr2 = _spmm(src, dst, edge_values, xw2)
    h1_2, m2 = _phase_a(xr2, w10t, w20t, adjt)
    item, hs = _phase_b2(h1_2, m2, xr2, item1, hn1)
    return (item[:N, :D], hs[:N, :D])


# speedup vs baseline: 5.9517x; 1.0000x over previous
import functools

import jax
import jax.numpy as jnp
from jax import lax
from jax.experimental import pallas as pl
from jax.experimental.pallas import tpu as pltpu
from jax.experimental.pallas import tpu_sc as plsc

N = 50000
E = 800000
D = 100
K = 100
DP = 128
NP = 50176
QR = 12544
STRIPE = QR // 16
EPT = E // 16
CH = 1024
NFULL = EPT // CH
NPAIR = NFULL // 2
TAIL = EPT - NFULL * CH
F = 80
CAP = F + 16

BN = 1024
GRID = NP // BN


def _spmm_body(src_hbm, dst_hbm, val_hbm, x_hbm, out_hbm,
               acc, sbufa, dbufa, vbufa, sbufb, dbufb, vbufb,
               cdst, crow, cval,
               dstf0, valf0, rowf0, rows0,
               dstf1, valf1, rowf1, rows1,
               ptr_ref, flg, sema, semb, gs0, gs1, ss0, ss1):
    core = lax.axis_index("c")
    sub = lax.axis_index("s")
    wid = sub * 2 + core
    iota16 = lax.iota(jnp.int32, 16)
    zero16 = jnp.zeros((16,), jnp.float32)

    e_base = sub * EPT
    sets = ((dstf0, valf0, rowf0, rows0, gs0, ss0),
            (dstf1, valf1, rowf1, rows1, gs1, ss1))

    def mul_rows(valf, rows):
        @pl.loop(0, F // 16)
        def _mul(rg):
            vv = valf[pl.ds(rg * 16, 16)]
            for j in range(16):
                v = jnp.full((16,), vv[j], jnp.float32)
                r = rg * 16 + j
                for u in range(8):
                    rows[r, pl.ds(u * 16, 16)] = rows[r, pl.ds(u * 16, 16)] * v

    def finish(t):
        dstf, valf, rowf, rows, gsem, ssem = sets[t]

        @pl.when(flg[1 + t] == 1)
        def _():
            pltpu.make_async_copy(x_hbm.at[dstf], rows, gsem).wait()
            mul_rows(valf, rows)
            pltpu.async_copy(rows, acc.at[rowf], ssem, add=True)
            flg[1 + t] = jnp.int32(0)
            flg[3 + t] = jnp.int32(1)

    def wait_scatter(t):
        dstf, valf, rowf, rows, gsem, ssem = sets[t]

        @pl.when(flg[3 + t] == 1)
        def _():
            pltpu.make_async_copy(rows, acc.at[rowf], ssem).wait()
            flg[3 + t] = jnp.int32(0)

    def start(t):
        dstf, valf, rowf, rows, gsem, ssem = sets[t]

        @pl.loop(0, F // 16)
        def _cp(k):
            dstf[pl.ds(k * 16, 16)] = cdst[pl.ds(k * 16, 16)]
            valf[pl.ds(k * 16, 16)] = cval[pl.ds(k * 16, 16)]
            rowf[pl.ds(k * 16, 16)] = crow[pl.ds(k * 16, 16)]
        pltpu.async_copy(x_hbm.at[dstf], rows, gsem)
        flg[1 + t] = jnp.int32(1)

    for q in range(2):
        Q = core * 2 + q
        qbase = Q * QR
        rbase = sub * STRIPE

        @pl.loop(0, F)
        def _zrow(r):
            for u in range(8):
                rows0[r, pl.ds(u * 16, 16)] = zero16
        for k in range(STRIPE // F):
            pltpu.sync_copy(rows0, acc.at[pl.ds(rbase + k * F, F)])
        rem = STRIPE % F
        if rem:
            pltpu.sync_copy(rows0.at[pl.ds(0, rem)],
                            acc.at[pl.ds(rbase + (STRIPE // F) * F, rem)])
        plsc.subcore_barrier()

        ptr_ref[0] = jnp.int32(0)
        for i in range(5):
            flg[i] = jnp.int32(0)

        def fire():
            a = flg[0]

            @pl.when(a == 0)
            def _():
                finish(1)
                wait_scatter(0)
                start(0)

            @pl.when(a == 1)
            def _():
                finish(0)
                wait_scatter(1)
                start(1)

            flg[0] = 1 - a
            for buf in (cdst, crow, cval):
                t_ = buf[pl.ds(F, 16)]
                buf[pl.ds(0, 16)] = t_

        def compact_groups(ngroups, sbuf, dbuf, vbuf):
            @pl.loop(0, ngroups)
            def group(g):
                sv = sbuf[pl.ds(g * 16, 16)]
                dv = dbuf[pl.ds(g * 16, 16)]
                vv = vbuf[pl.ds(g * 16, 16)]
                qv = ((sv >= QR).astype(jnp.int32)
                      + (sv >= 2 * QR).astype(jnp.int32)
                      + (sv >= 3 * QR).astype(jnp.int32))
                m = qv == Q
                rv = sv - qbase
                p = ptr_ref[0]
                mi = m.astype(jnp.int32)
                pos = p + plsc.cumsum(mi) - 1
                plsc.store_scatter(cdst, [pos], dv, mask=m)
                plsc.store_scatter(crow, [pos], rv, mask=m)
                plsc.store_scatter(cval, [pos], vv, mask=m)
                p2 = p + jnp.sum(mi)

                @pl.when(p2 >= F)
                def _():
                    fire()

                ptr_ref[0] = jnp.where(p2 >= F, p2 - F, p2)

        def issue(off, sb, db, vb, sem):
            pltpu.async_copy(src_hbm.at[pl.ds(off, CH)], sb, sem)
            pltpu.async_copy(dst_hbm.at[pl.ds(off, CH)], db, sem)
            pltpu.async_copy(val_hbm.at[pl.ds(off, CH)], vb, sem)

        def drain(sb, db, vb, sem):
            pltpu.make_async_copy(src_hbm.at[pl.ds(0, CH)], sb, sem).wait()
            pltpu.make_async_copy(dst_hbm.at[pl.ds(0, CH)], db, sem).wait()
            pltpu.make_async_copy(val_hbm.at[pl.ds(0, CH)], vb, sem).wait()

        issue(e_base, sbufa, dbufa, vbufa, sema)

        @pl.loop(0, NPAIR)
        def _pair(k):
            ci = k * 2
            issue(e_base + (ci + 1) * CH, sbufb, dbufb, vbufb, semb)
            drain(sbufa, dbufa, vbufa, sema)
            compact_groups(CH // 16, sbufa, dbufa, vbufa)

            @pl.when(k < NPAIR - 1)
            def _():
                issue(e_base + (ci + 2) * CH, sbufa, dbufa, vbufa, sema)

            drain(sbufb, dbufb, vbufb, semb)
            compact_groups(CH // 16, sbufb, dbufb, vbufb)

        off = e_base + NFULL * CH
        pltpu.sync_copy(src_hbm.at[pl.ds(off, TAIL)], sbufa.at[pl.ds(0, TAIL)])
        pltpu.sync_copy(dst_hbm.at[pl.ds(off, TAIL)], dbufa.at[pl.ds(0, TAIL)])
        pltpu.sync_copy(val_hbm.at[pl.ds(off, TAIL)], vbufa.at[pl.ds(0, TAIL)])
        compact_groups(TAIL // 16, sbufa, dbufa, vbufa)

        finish(0)
        finish(1)
        wait_scatter(0)
        wait_scatter(1)

        @pl.loop(0, F // 16)
        def _san(g):
            lane = g * 16 + iota16
            m = lane < ptr_ref[0]
            safe = wid * 16 + iota16
            dvv = cdst[pl.ds(g * 16, 16)]
            rvv = crow[pl.ds(g * 16, 16)]
            vvv = cval[pl.ds(g * 16, 16)]
            cdst[pl.ds(g * 16, 16)] = jnp.where(m, dvv, safe)
            crow[pl.ds(g * 16, 16)] = jnp.where(m, rvv, 0)
            cval[pl.ds(g * 16, 16)] = jnp.where(m, vvv, 0.0)
        start(0)
        finish(0)
        wait_scatter(0)

        plsc.subcore_barrier()
        pltpu.sync_copy(acc.at[pl.ds(rbase, STRIPE)],
                        out_hbm.at[pl.ds(qbase + rbase, STRIPE)])
        plsc.subcore_barrier()


def _spmm(src, dst, val, xpad):
    mesh = plsc.VectorSubcoreMesh(core_axis_name="c", subcore_axis_name="s")
    f = pl.kernel(
        _spmm_body,
        out_type=jax.ShapeDtypeStruct((NP, DP), jnp.float32),
        mesh=mesh,
        compiler_params=pltpu.CompilerParams(needs_layout_passes=False, use_tc_tiling_on_sc=True),
        scratch_types=[
            pltpu.VMEM_SHARED((QR, DP), jnp.float32),
            pltpu.VMEM((CH,), jnp.int32),
            pltpu.VMEM((CH,), jnp.int32),
            pltpu.VMEM((CH,), jnp.float32),
            pltpu.VMEM((CH,), jnp.int32),
            pltpu.VMEM((CH,), jnp.int32),
            pltpu.VMEM((CH,), jnp.float32),
            pltpu.VMEM((CAP,), jnp.int32),
            pltpu.VMEM((CAP,), jnp.int32),
            pltpu.VMEM((CAP,), jnp.float32),
            pltpu.VMEM((F,), jnp.int32),
            pltpu.VMEM((F,), jnp.float32),
            pltpu.VMEM((F,), jnp.int32),
            pltpu.VMEM((F, DP), jnp.float32),
            pltpu.VMEM((F,), jnp.int32),
            pltpu.VMEM((F,), jnp.float32),
            pltpu.VMEM((F,), jnp.int32),
            pltpu.VMEM((F, DP), jnp.float32),
            pltpu.SMEM((1,), jnp.int32),
            pltpu.SMEM((5,), jnp.int32),
            pltpu.SemaphoreType.DMA,
            pltpu.SemaphoreType.DMA,
            pltpu.SemaphoreType.DMA,
            pltpu.SemaphoreType.DMA,
            pltpu.SemaphoreType.DMA,
            pltpu.SemaphoreType.DMA,
        ],
    )
    return f(src, dst, val, xpad)


def _l2n(x):
    n = jnp.sqrt(jnp.sum(x * x, axis=1, keepdims=True))
    return x / jnp.maximum(n, 1e-12)


def _linear_body(x_ref, w_ref, o_ref):
    o_ref[...] = jnp.dot(x_ref[...], w_ref[...],
                         preferred_element_type=jnp.float32)


def _linear(x, wt):
    return pl.pallas_call(
        _linear_body,
        grid=(GRID,),
        in_specs=[pl.BlockSpec((BN, DP), lambda i: (i, 0)),
                  pl.BlockSpec((DP, DP), lambda i: (0, 0))],
        out_specs=pl.BlockSpec((BN, DP), lambda i: (i, 0)),
        out_shape=jax.ShapeDtypeStruct((NP, DP), jnp.float32),
    )(x, wt)


def _phase_a_body(xr_ref, w10t_ref, w20t_ref, adjt_ref, h1_ref, m_ref, macc_ref):
    i = pl.program_id(0)
    x = xr_ref[...]
    t = jnp.maximum(jnp.dot(x, w10t_ref[...],
                            preferred_element_type=jnp.float32), 0.0)
    logits = jnp.dot(t, w20t_ref[...], preferred_element_type=jnp.float32)
    lanes = lax.broadcasted_iota(jnp.int32, (1, DP), 1)
    logits = jnp.where(lanes < K, logits, -1e30)
    mx = jnp.max(logits, axis=1, keepdims=True)
    e = jnp.exp(logits - mx)
    h1 = e / jnp.sum(e, axis=1, keepdims=True)
    h1_ref[...] = h1
    a = h1 * adjt_ref[...]
    s = jnp.sum(a, axis=1, keepdims=True)
    g = a / jnp.maximum(s, 1e-30)
    mblk = lax.dot_general(g, x, (((0,), (0,)), ((), ())),
                           preferred_element_type=jnp.float32)

    @pl.when(i == 0)
    def _():
        macc_ref[...] = mblk

    @pl.when(i > 0)
    def _():
        macc_ref[...] += mblk

    @pl.when(i == GRID - 1)
    def _():
        m_ref[...] = macc_ref[...]


def _phase_a(xr, w10t, w20t, adjt):
    return pl.pallas_call(
        _phase_a_body,
        grid=(GRID,),
        in_specs=[pl.BlockSpec((BN, DP), lambda i: (i, 0)),
                  pl.BlockSpec((DP, DP), lambda i: (0, 0)),
                  pl.BlockSpec((DP, DP), lambda i: (0, 0)),
                  pl.BlockSpec((BN, DP), lambda i: (i, 0))],
        out_specs=[pl.BlockSpec((BN, DP), lambda i: (i, 0)),
                   pl.BlockSpec((DP, DP), lambda i: (0, 0))],
        out_shape=[jax.ShapeDtypeStruct((NP, DP), jnp.float32),
                   jax.ShapeDtypeStruct((DP, DP), jnp.float32)],
        scratch_shapes=[pltpu.VMEM((DP, DP), jnp.float32)],
    )(xr, w10t, w20t, adjt)


def _phase_b1_body(h1_ref, m_ref, xr_ref, emb_ref, w1t_ref,
                   item1_ref, hn1_ref, xw2_ref):
    h = jnp.dot(h1_ref[...], m_ref[...], preferred_element_type=jnp.float32)
    x1 = h + xr_ref[...]
    item1_ref[...] = emb_ref[...] + _l2n(x1)
    hn1_ref[...] = _l2n(h)
    xw2_ref[...] = jnp.dot(x1, w1t_ref[...], preferred_element_type=jnp.float32)


def _phase_b1(h1, m, xr, emb, w1t):
    blk = pl.BlockSpec((BN, DP), lambda i: (i, 0))
    return pl.pallas_call(
        _phase_b1_body,
        grid=(GRID,),
        in_specs=[blk, pl.BlockSpec((DP, DP), lambda i: (0, 0)), blk, blk,
                  pl.BlockSpec((DP, DP), lambda i: (0, 0))],
        out_specs=[blk, blk, blk],
        out_shape=[jax.ShapeDtypeStruct((NP, DP), jnp.float32)] * 3,
    )(h1, m, xr, emb, w1t)


def _phase_b2_body(h1_ref, m_ref, xr_ref, item1_ref, hn1_ref,
                   item_ref, hs_ref):
    h = jnp.dot(h1_ref[...], m_ref[...], preferred_element_type=jnp.float32)
    x2 = h + xr_ref[...]
    item_ref[...] = (item1_ref[...] + _l2n(x2)) * (1.0 / 3.0)
    hs_ref[...] = (hn1_ref[...] + _l2n(h)) * 0.5


def _phase_b2(h1, m, xr, item1, hn1):
    blk = pl.BlockSpec((BN, DP), lambda i: (i, 0))
    return pl.pallas_call(
        _phase_b2_body,
        grid=(GRID,),
        in_specs=[blk, pl.BlockSpec((DP, DP), lambda i: (0, 0)), blk, blk, blk],
        out_specs=[blk, blk],
        out_shape=[jax.ShapeDtypeStruct((NP, DP), jnp.float32)] * 2,
    )(h1, m, xr, item1, hn1)


def _pad2(a, rows, cols):
    return jnp.zeros((rows, cols), a.dtype).at[:a.shape[0], :a.shape[1]].set(a)


def kernel(embedding, edge_index, edge_values, adj, W_item0, W_item1,
           W_i10, W_i20, channel):
    src = edge_index[0]
    dst = edge_index[1]
    emb = _pad2(embedding, NP, DP)
    adjt = _pad2(adj.T, NP, DP)
    w0t = _pad2(W_item0.T, DP, DP)
    w1t = _pad2(W_item1.T, DP, DP)
    w10t = _pad2(W_i10.T + jnp.eye(D, dtype=jnp.float32), DP, DP)
    w20t = _pad2(W_i20.T, DP, DP)

    xw1 = _linear(emb, w0t)
    xr1 = _spmm(src, dst, edge_values, xw1)
    h1_1, m1 = _phase_a(xr1, w10t, w20t, adjt)
    item1, hn1, xw2 = _phase_b1(h1_1, m1, xr1, emb, w1t)
    xr2 = _spmm(src, dst, edge_values, xw2)
    h1_2, m2 = _phase_a(xr2, w10t, w20t, adjt)
    item, hs = _phase_b2(h1_2, m2, xr2, item1, hn1)
    return (item[:N, :D], hs[:N, :D])


# speedup vs baseline: 6.1696x; 1.0366x over previous
import functools

import jax
import jax.numpy as jnp
from jax import lax
from jax.experimental import pallas as pl
from jax.experimental.pallas import tpu as pltpu
from jax.experimental.pallas import tpu_sc as plsc

N = 50000
E = 800000
D = 100
K = 100
DP = 128
NP = 50176
QR = 12544
STRIPE = QR // 16
EPT = E // 16
CH = 1024
NFULL = EPT // CH
NPAIR = NFULL // 2
TAIL = EPT - NFULL * CH
F = 80
CAP = F + 16

BN = 1792
GRID = NP // BN


def _spmm_body(src_hbm, dst_hbm, val_hbm, x_hbm, out_hbm,
               acc, sbufa, dbufa, vbufa, sbufb, dbufb, vbufb,
               cdst, crow, cval,
               dstf0, valf0, rowf0, rows0,
               dstf1, valf1, rowf1, rows1,
               ptr_ref, flg, sema, semb, gs0, gs1, ss0, ss1):
    core = lax.axis_index("c")
    sub = lax.axis_index("s")
    wid = sub * 2 + core
    iota16 = lax.iota(jnp.int32, 16)
    zero16 = jnp.zeros((16,), jnp.float32)

    e_base = sub * EPT
    sets = ((dstf0, valf0, rowf0, rows0, gs0, ss0),
            (dstf1, valf1, rowf1, rows1, gs1, ss1))

    def mul_rows(valf, rows):
        @pl.loop(0, F // 16)
        def _mul(rg):
            vv = valf[pl.ds(rg * 16, 16)]
            for j in range(16):
                v = jnp.full((16,), vv[j], jnp.float32)
                r = rg * 16 + j
                for u in range(8):
                    rows[r, pl.ds(u * 16, 16)] = rows[r, pl.ds(u * 16, 16)] * v

    def finish(t):
        dstf, valf, rowf, rows, gsem, ssem = sets[t]

        @pl.when(flg[1 + t] == 1)
        def _():
            pltpu.make_async_copy(x_hbm.at[dstf], rows, gsem).wait()
            mul_rows(valf, rows)
            pltpu.async_copy(rows, acc.at[rowf], ssem, add=True)
            flg[1 + t] = jnp.int32(0)
            flg[3 + t] = jnp.int32(1)

    def wait_scatter(t):
        dstf, valf, rowf, rows, gsem, ssem = sets[t]

        @pl.when(flg[3 + t] == 1)
        def _():
            pltpu.make_async_copy(rows, acc.at[rowf], ssem).wait()
            flg[3 + t] = jnp.int32(0)

    def start(t):
        dstf, valf, rowf, rows, gsem, ssem = sets[t]

        @pl.loop(0, F // 16)
        def _cp(k):
            dstf[pl.ds(k * 16, 16)] = cdst[pl.ds(k * 16, 16)]
            valf[pl.ds(k * 16, 16)] = cval[pl.ds(k * 16, 16)]
            rowf[pl.ds(k * 16, 16)] = crow[pl.ds(k * 16, 16)]
        pltpu.async_copy(x_hbm.at[dstf], rows, gsem)
        flg[1 + t] = jnp.int32(1)

    for q in range(2):
        Q = core * 2 + q
        qbase = Q * QR
        rbase = sub * STRIPE

        @pl.loop(0, F)
        def _zrow(r):
            for u in range(8):
                rows0[r, pl.ds(u * 16, 16)] = zero16
        for k in range(STRIPE // F):
            pltpu.sync_copy(rows0, acc.at[pl.ds(rbase + k * F, F)])
        rem = STRIPE % F
        if rem:
            pltpu.sync_copy(rows0.at[pl.ds(0, rem)],
                            acc.at[pl.ds(rbase + (STRIPE // F) * F, rem)])
        plsc.subcore_barrier()

        ptr_ref[0] = jnp.int32(0)
        for i in range(5):
            flg[i] = jnp.int32(0)

        def fire():
            a = flg[0]

            @pl.when(a == 0)
            def _():
                finish(1)
                wait_scatter(0)
                start(0)

            @pl.when(a == 1)
            def _():
                finish(0)
                wait_scatter(1)
                start(1)

            flg[0] = 1 - a
            for buf in (cdst, crow, cval):
                t_ = buf[pl.ds(F, 16)]
                buf[pl.ds(0, 16)] = t_

        def compact_groups(ngroups, sbuf, dbuf, vbuf):
            @pl.loop(0, ngroups)
            def group(g):
                sv = sbuf[pl.ds(g * 16, 16)]
                dv = dbuf[pl.ds(g * 16, 16)]
                vv = vbuf[pl.ds(g * 16, 16)]
                qv = ((sv >= QR).astype(jnp.int32)
                      + (sv >= 2 * QR).astype(jnp.int32)
                      + (sv >= 3 * QR).astype(jnp.int32))
                m = qv == Q
                rv = sv - qbase
                p = ptr_ref[0]
                mi = m.astype(jnp.int32)
                pos = p + plsc.cumsum(mi) - 1
                plsc.store_scatter(cdst, [pos], dv, mask=m)
                plsc.store_scatter(crow, [pos], rv, mask=m)
                plsc.store_scatter(cval, [pos], vv, mask=m)
                p2 = p + jnp.sum(mi)

                @pl.when(p2 >= F)
                def _():
                    fire()

                ptr_ref[0] = jnp.where(p2 >= F, p2 - F, p2)

        def issue(off, sb, db, vb, sem):
            pltpu.async_copy(src_hbm.at[pl.ds(off, CH)], sb, sem)
            pltpu.async_copy(dst_hbm.at[pl.ds(off, CH)], db, sem)
            pltpu.async_copy(val_hbm.at[pl.ds(off, CH)], vb, sem)

        def drain(sb, db, vb, sem):
            pltpu.make_async_copy(src_hbm.at[pl.ds(0, CH)], sb, sem).wait()
            pltpu.make_async_copy(dst_hbm.at[pl.ds(0, CH)], db, sem).wait()
            pltpu.make_async_copy(val_hbm.at[pl.ds(0, CH)], vb, sem).wait()

        issue(e_base, sbufa, dbufa, vbufa, sema)

        @pl.loop(0, NPAIR)
        def _pair(k):
            ci = k * 2
            issue(e_base + (ci + 1) * CH, sbufb, dbufb, vbufb, semb)
            drain(sbufa, dbufa, vbufa, sema)
            compact_groups(CH // 16, sbufa, dbufa, vbufa)

            @pl.when(k < NPAIR - 1)
            def _():
                issue(e_base + (ci + 2) * CH, sbufa, dbufa, vbufa, sema)

            drain(sbufb, dbufb, vbufb, semb)
            compact_groups(CH // 16, sbufb, dbufb, vbufb)

        off = e_base + NFULL * CH
        pltpu.sync_copy(src_hbm.at[pl.ds(off, TAIL)], sbufa.at[pl.ds(0, TAIL)])
        pltpu.sync_copy(dst_hbm.at[pl.ds(off, TAIL)], dbufa.at[pl.ds(0, TAIL)])
        pltpu.sync_copy(val_hbm.at[pl.ds(off, TAIL)], vbufa.at[pl.ds(0, TAIL)])
        compact_groups(TAIL // 16, sbufa, dbufa, vbufa)

        finish(0)
        finish(1)
        wait_scatter(0)
        wait_scatter(1)

        @pl.loop(0, F // 16)
        def _san(g):
            lane = g * 16 + iota16
            m = lane < ptr_ref[0]
            safe = wid * 16 + iota16
            dvv = cdst[pl.ds(g * 16, 16)]
            rvv = crow[pl.ds(g * 16, 16)]
            vvv = cval[pl.ds(g * 16, 16)]
            cdst[pl.ds(g * 16, 16)] = jnp.where(m, dvv, safe)
            crow[pl.ds(g * 16, 16)] = jnp.where(m, rvv, 0)
            cval[pl.ds(g * 16, 16)] = jnp.where(m, vvv, 0.0)
        start(0)
        finish(0)
        wait_scatter(0)

        plsc.subcore_barrier()
        pltpu.sync_copy(acc.at[pl.ds(rbase, STRIPE)],
                        out_hbm.at[pl.ds(qbase + rbase, STRIPE)])
        plsc.subcore_barrier()


def _spmm(src, dst, val, xpad):
    mesh = plsc.VectorSubcoreMesh(core_axis_name="c", subcore_axis_name="s")
    f = pl.kernel(
        _spmm_body,
        out_type=jax.ShapeDtypeStruct((NP, DP), jnp.float32),
        mesh=mesh,
        compiler_params=pltpu.CompilerParams(needs_layout_passes=False),
        scratch_types=[
            pltpu.VMEM_SHARED((QR, DP), jnp.float32),
            pltpu.VMEM((CH,), jnp.int32),
            pltpu.VMEM((CH,), jnp.int32),
            pltpu.VMEM((CH,), jnp.float32),
            pltpu.VMEM((CH,), jnp.int32),
            pltpu.VMEM((CH,), jnp.int32),
            pltpu.VMEM((CH,), jnp.float32),
            pltpu.VMEM((CAP,), jnp.int32),
            pltpu.VMEM((CAP,), jnp.int32),
            pltpu.VMEM((CAP,), jnp.float32),
            pltpu.VMEM((F,), jnp.int32),
            pltpu.VMEM((F,), jnp.float32),
            pltpu.VMEM((F,), jnp.int32),
            pltpu.VMEM((F, DP), jnp.float32),
            pltpu.VMEM((F,), jnp.int32),
            pltpu.VMEM((F,), jnp.float32),
            pltpu.VMEM((F,), jnp.int32),
            pltpu.VMEM((F, DP), jnp.float32),
            pltpu.SMEM((1,), jnp.int32),
            pltpu.SMEM((5,), jnp.int32),
            pltpu.SemaphoreType.DMA,
            pltpu.SemaphoreType.DMA,
            pltpu.SemaphoreType.DMA,
            pltpu.SemaphoreType.DMA,
            pltpu.SemaphoreType.DMA,
            pltpu.SemaphoreType.DMA,
        ],
    )
    return f(src, dst, val, xpad)


def _l2n(x):
    n = jnp.sqrt(jnp.sum(x * x, axis=1, keepdims=True))
    return x / jnp.maximum(n, 1e-12)


def _linear_body(x_ref, w_ref, o_ref):
    o_ref[...] = jnp.dot(x_ref[...], w_ref[...],
                         preferred_element_type=jnp.float32)


def _linear(x, wt):
    return pl.pallas_call(
        _linear_body,
        grid=(GRID,),
        in_specs=[pl.BlockSpec((BN, DP), lambda i: (i, 0)),
                  pl.BlockSpec((DP, DP), lambda i: (0, 0))],
        out_specs=pl.BlockSpec((BN, DP), lambda i: (i, 0)),
        out_shape=jax.ShapeDtypeStruct((NP, DP), jnp.float32),
    )(x, wt)


def _phase_a_body(xr_ref, w10t_ref, w20t_ref, adjt_ref, h1_ref, m_ref, macc_ref):
    i = pl.program_id(0)
    x = xr_ref[...]
    t = jnp.maximum(jnp.dot(x, w10t_ref[...],
                            preferred_element_type=jnp.float32), 0.0)
    logits = jnp.dot(t, w20t_ref[...], preferred_element_type=jnp.float32)
    lanes = lax.broadcasted_iota(jnp.int32, (1, DP), 1)
    logits = jnp.where(lanes < K, logits, -1e30)
    mx = jnp.max(logits, axis=1, keepdims=True)
    e = jnp.exp(logits - mx)
    h1 = e / jnp.sum(e, axis=1, keepdims=True)
    h1_ref[...] = h1
    a = h1 * adjt_ref[...]
    s = jnp.sum(a, axis=1, keepdims=True)
    g = a / jnp.maximum(s, 1e-30)
    mblk = lax.dot_general(g, x, (((0,), (0,)), ((), ())),
                           preferred_element_type=jnp.float32)

    @pl.when(i == 0)
    def _():
        macc_ref[...] = mblk

    @pl.when(i > 0)
    def _():
        macc_ref[...] += mblk

    @pl.when(i == GRID - 1)
    def _():
        m_ref[...] = macc_ref[...]


def _phase_a(xr, w10t, w20t, adjt):
    return pl.pallas_call(
        _phase_a_body,
        grid=(GRID,),
        in_specs=[pl.BlockSpec((BN, DP), lambda i: (i, 0)),
                  pl.BlockSpec((DP, DP), lambda i: (0, 0)),
                  pl.BlockSpec((DP, DP), lambda i: (0, 0)),
                  pl.BlockSpec((BN, DP), lambda i: (i, 0))],
        out_specs=[pl.BlockSpec((BN, DP), lambda i: (i, 0)),
                   pl.BlockSpec((DP, DP), lambda i: (0, 0))],
        out_shape=[jax.ShapeDtypeStruct((NP, DP), jnp.float32),
                   jax.ShapeDtypeStruct((DP, DP), jnp.float32)],
        scratch_shapes=[pltpu.VMEM((DP, DP), jnp.float32)],
    )(xr, w10t, w20t, adjt)


def _phase_b1_body(h1_ref, m_ref, xr_ref, emb_ref, w1t_ref,
                   item1_ref, hn1_ref, xw2_ref):
    h = jnp.dot(h1_ref[...], m_ref[...], preferred_element_type=jnp.float32)
    x1 = h + xr_ref[...]
    item1_ref[...] = emb_ref[...] + _l2n(x1)
    hn1_ref[...] = _l2n(h)
    xw2_ref[...] = jnp.dot(x1, w1t_ref[...], preferred_element_type=jnp.float32)


def _phase_b1(h1, m, xr, emb, w1t):
    blk = pl.BlockSpec((BN, DP), lambda i: (i, 0))
    return pl.pallas_call(
        _phase_b1_body,
        grid=(GRID,),
        in_specs=[blk, pl.BlockSpec((DP, DP), lambda i: (0, 0)), blk, blk,
                  pl.BlockSpec((DP, DP), lambda i: (0, 0))],
        out_specs=[blk, blk, blk],
        out_shape=[jax.ShapeDtypeStruct((NP, DP), jnp.float32)] * 3,
    )(h1, m, xr, emb, w1t)


def _phase_b2_body(h1_ref, m_ref, xr_ref, item1_ref, hn1_ref,
                   item_ref, hs_ref):
    h = jnp.dot(h1_ref[...], m_ref[...], preferred_element_type=jnp.float32)
    x2 = h + xr_ref[...]
    item_ref[...] = (item1_ref[...] + _l2n(x2)) * (1.0 / 3.0)
    hs_ref[...] = (hn1_ref[...] + _l2n(h)) * 0.5


def _phase_b2(h1, m, xr, item1, hn1):
    blk = pl.BlockSpec((BN, DP), lambda i: (i, 0))
    return pl.pallas_call(
        _phase_b2_body,
        grid=(GRID,),
        in_specs=[blk, pl.BlockSpec((DP, DP), lambda i: (0, 0)), blk, blk, blk],
        out_specs=[blk, blk],
        out_shape=[jax.ShapeDtypeStruct((NP, DP), jnp.float32)] * 2,
    )(h1, m, xr, item1, hn1)


def _pad2(a, rows, cols):
    return jnp.zeros((rows, cols), a.dtype).at[:a.shape[0], :a.shape[1]].set(a)


def kernel(embedding, edge_index, edge_values, adj, W_item0, W_item1,
           W_i10, W_i20, channel):
    src = edge_index[0]
    dst = edge_index[1]
    emb = _pad2(embedding, NP, DP)
    adjt = _pad2(adj.T, NP, DP)
    w0t = _pad2(W_item0.T, DP, DP)
    w1t = _pad2(W_item1.T, DP, DP)
    w10t = _pad2(W_i10.T + jnp.eye(D, dtype=jnp.float32), DP, DP)
    w20t = _pad2(W_i20.T, DP, DP)

    xw1 = _linear(emb, w0t)
    xr1 = _spmm(src, dst, edge_values, xw1)
    h1_1, m1 = _phase_a(xr1, w10t, w20t, adjt)
    item1, hn1, xw2 = _phase_b1(h1_1, m1, xr1, emb, w1t)
    xr2 = _spmm(src, dst, edge_values, xw2)
    h1_2, m2 = _phase_a(xr2, w10t, w20t, adjt)
    item, hs = _phase_b2(h1_2, m2, xr2, item1, hn1)
    return (item[:N, :D], hs[:N, :D])
